# Initial kernel scaffold; baseline (speedup 1.0000x reference)
import jax
import jax.numpy as jnp
from jax.experimental import pallas as pl


def kernel(xyz, cls_label, npoint):
    raise NotImplementedError("write your pallas kernel here")


# speedup vs baseline: 6.2638x; 6.2638x over previous
import jax
import jax.numpy as jnp
from jax import lax
from jax.experimental import pallas as pl
from jax.experimental.pallas import tpu as pltpu
from jax.experimental.pallas import tpu_sc as plsc

L = 16
B = 8
N = 4096
NPOINT = 512
NCHUNK = N // L
RADIUS_SQ = 0.25
BIG = 2 ** 30


def _bf16_round(x):
    u = plsc.bitcast(x, jnp.uint32)
    u = (u + jnp.uint32(0x7FFF) + ((u >> 16) & jnp.uint32(1))) & jnp.uint32(0xFFFF0000)
    return plsc.bitcast(u, jnp.float32)


def _store_scalar_i32(ref, pos, val):
    lane0 = lax.iota(jnp.int32, L) == 0
    plsc.store_scatter(
        ref,
        [jnp.full((L,), pos, jnp.int32)],
        jnp.full((L,), val, jnp.int32),
        mask=lane0,
    )


def _sc_body(xyz_hbm, out_hbm, x_v, dist_v, cent_v, sb_v, idx_v, xb_v):
    cid = lax.axis_index("c")
    sid = lax.axis_index("s")
    wid = sid * 2 + cid
    lanes = lax.iota(jnp.int32, L)

    @pl.when(wid < B)
    def _():
        b = wid
        pltpu.sync_copy(xyz_hbm.at[b], x_v)

        zeros16 = jnp.zeros((L,), jnp.int32)
        ones16 = jnp.full((L,), 1, jnp.int32)
        twos16 = jnp.full((L,), 2, jnp.int32)

        def init_chunk(ci, carry):
            rmax, ridx = carry
            off = ci * L
            xv = x_v[0, pl.ds(off, L)]
            dist_v[pl.ds(off, L)] = jnp.full((L,), 1e10, jnp.float32)
            m = xv > rmax
            rmax = jnp.where(m, xv, rmax)
            ridx = jnp.where(m, off + lanes, ridx)
            return rmax, ridx

        rmax0 = jnp.full((L,), -1e30, jnp.float32)
        rmax, ridx = lax.fori_loop(0, NCHUNK, init_chunk, (rmax0, jnp.zeros((L,), jnp.int32)))
        mx = jnp.max(rmax)
        far0 = jnp.min(jnp.where(rmax == mx, ridx, BIG))

        def fps_step(i, far):
            _store_scalar_i32(cent_v, i, far)
            farv = jnp.full((L,), far, jnp.int32)
            cx = plsc.load_gather(x_v, [zeros16, farv])
            cy = plsc.load_gather(x_v, [ones16, farv])
            cz = plsc.load_gather(x_v, [twos16, farv])

            def chunk(ci, carry):
                rmax, ridx = carry
                off = ci * L
                xv = x_v[0, pl.ds(off, L)]
                yv = x_v[1, pl.ds(off, L)]
                zv = x_v[2, pl.ds(off, L)]
                dx = xv - cx
                dy = yv - cy
                dz = zv - cz
                d = dx * dx + dy * dy + dz * dz
                old = dist_v[pl.ds(off, L)]
                nd = jnp.where(d < old, d, old)
                dist_v[pl.ds(off, L)] = nd
                m = nd > rmax
                rmax = jnp.where(m, nd, rmax)
                ridx = jnp.where(m, off + lanes, ridx)
                return rmax, ridx

            rmax, ridx = lax.fori_loop(
                0, NCHUNK, chunk,
                (jnp.full((L,), -1.0, jnp.float32), jnp.zeros((L,), jnp.int32)))
            mx = jnp.max(rmax)
            return jnp.min(jnp.where(rmax == mx, ridx, BIG))

        lax.fori_loop(0, NPOINT, fps_step, far0)

        def sb_chunk(ci, _):
            off = ci * L
            xv = x_v[0, pl.ds(off, L)]
            yv = x_v[1, pl.ds(off, L)]
            zv = x_v[2, pl.ds(off, L)]
            sb_v[pl.ds(off, L)] = xv * xv + yv * yv + zv * zv
            xb_v[0, pl.ds(off, L)] = _bf16_round(xv)
            xb_v[1, pl.ds(off, L)] = _bf16_round(yv)
            xb_v[2, pl.ds(off, L)] = _bf16_round(zv)
            return 0

        lax.fori_loop(0, NCHUNK, sb_chunk, 0)

        def ball(si, _):
            fsplat = plsc.load_gather(cent_v, [jnp.full((L,), si, jnp.int32)])
            cx = plsc.load_gather(x_v, [zeros16, fsplat])
            cy = plsc.load_gather(x_v, [ones16, fsplat])
            cz = plsc.load_gather(x_v, [twos16, fsplat])
            sa = cx * cx + cy * cy + cz * cz
            cxb = plsc.load_gather(xb_v, [zeros16, fsplat])
            cyb = plsc.load_gather(xb_v, [ones16, fsplat])
            czb = plsc.load_gather(xb_v, [twos16, fsplat])

            def cond(carry):
                ci, found = carry
                return jnp.logical_and(found >= BIG, ci < NCHUNK)

            def body(carry):
                ci, found = carry
                off = ci * L
                xv = xb_v[0, pl.ds(off, L)]
                yv = xb_v[1, pl.ds(off, L)]
                zv = xb_v[2, pl.ds(off, L)]
                sbv = sb_v[pl.ds(off, L)]
                dot = cxb * xv + cyb * yv + czb * zv
                t = -2.0 * dot
                t = t + sa
                t = t + sbv
                hit = jnp.logical_not(t > RADIUS_SQ)
                cand = jnp.min(jnp.where(hit, off + lanes, BIG))
                return ci + 1, jnp.minimum(found, cand)

            _, found = lax.while_loop(cond, body, (jnp.int32(0), BIG))
            _store_scalar_i32(idx_v, si, found)
            return 0

        lax.fori_loop(0, NPOINT, ball, 0)
        pltpu.sync_copy(idx_v, out_hbm.at[b])


_mesh = plsc.VectorSubcoreMesh(core_axis_name="c", subcore_axis_name="s")

_sc_call = pl.kernel(
    _sc_body,
    out_type=jax.ShapeDtypeStruct((B, NPOINT), jnp.int32),
    mesh=_mesh,
    compiler_params=pltpu.CompilerParams(needs_layout_passes=False),
    scratch_types=[
        pltpu.VMEM((3, N), jnp.float32),
        pltpu.VMEM((N,), jnp.float32),
        pltpu.VMEM((NPOINT,), jnp.int32),
        pltpu.VMEM((N,), jnp.float32),
        pltpu.VMEM((NPOINT,), jnp.int32),
        pltpu.VMEM((3, N), jnp.float32),
    ],
)


def kernel(xyz, cls_label, npoint):
    del cls_label, npoint
    out = _sc_call(xyz)
    return out.reshape(B, NPOINT, 1)


# speedup vs baseline: 20.8923x; 3.3354x over previous
import jax
import jax.numpy as jnp
from jax import lax
from jax.experimental import pallas as pl
from jax.experimental.pallas import tpu as pltpu
from jax.experimental.pallas import tpu_sc as plsc

L = 16
B = 8
N = 4096
NPOINT = 512
NCHUNK = N // L
RADIUS_SQ = 0.25
BIG = 2 ** 30


def _bf16_round(x):
    u = plsc.bitcast(x, jnp.uint32)
    u = (u + jnp.uint32(0x7FFF) + ((u >> 16) & jnp.uint32(1))) & jnp.uint32(0xFFFF0000)
    return plsc.bitcast(u, jnp.float32)


def _store_scalar_i32(ref, pos, val):
    lane0 = lax.iota(jnp.int32, L) == 0
    plsc.store_scatter(
        ref,
        [jnp.full((L,), pos, jnp.int32)],
        jnp.full((L,), val, jnp.int32),
        mask=lane0,
    )


def _sc_body(xyz_hbm, out_hbm, x_v, dist_v, cent_v, sb_v, idx_v, xb_v):
    cid = lax.axis_index("c")
    sid = lax.axis_index("s")
    wid = sid * 2 + cid
    lanes = lax.iota(jnp.int32, L)

    @pl.when(wid < B)
    def _():
        b = wid
        pltpu.sync_copy(xyz_hbm.at[b], x_v)

        zeros16 = jnp.zeros((L,), jnp.int32)
        ones16 = jnp.full((L,), 1, jnp.int32)
        twos16 = jnp.full((L,), 2, jnp.int32)

        rmax0 = jnp.full((L,), -1e30, jnp.float32)

        @plsc.parallel_loop(0, NCHUNK, unroll=8,
                            carry=(rmax0, jnp.zeros((L,), jnp.int32)))
        def init_chunk(ci, carry):
            rmax, ridx = carry
            off = ci * L
            xv = x_v[0, pl.ds(off, L)]
            dist_v[pl.ds(off, L)] = jnp.full((L,), 1e10, jnp.float32)
            m = xv > rmax
            rmax = jnp.where(m, xv, rmax)
            ridx = jnp.where(m, off + lanes, ridx)
            return rmax, ridx

        rmax, ridx = init_chunk
        mx = jnp.max(rmax)
        far0 = jnp.min(jnp.where(rmax == mx, ridx, BIG))

        def fps_step(i, far):
            _store_scalar_i32(cent_v, i, far)
            farv = jnp.full((L,), far, jnp.int32)
            cx = plsc.load_gather(x_v, [zeros16, farv])
            cy = plsc.load_gather(x_v, [ones16, farv])
            cz = plsc.load_gather(x_v, [twos16, farv])

            @plsc.parallel_loop(0, NCHUNK, unroll=8,
                                carry=(jnp.full((L,), -1.0, jnp.float32),
                                       jnp.zeros((L,), jnp.int32)))
            def chunk(ci, carry):
                rmax, ridx = carry
                off = ci * L
                xv = x_v[0, pl.ds(off, L)]
                yv = x_v[1, pl.ds(off, L)]
                zv = x_v[2, pl.ds(off, L)]
                dx = xv - cx
                dy = yv - cy
                dz = zv - cz
                d = dx * dx + dy * dy + dz * dz
                old = dist_v[pl.ds(off, L)]
                nd = jnp.where(d < old, d, old)
                dist_v[pl.ds(off, L)] = nd
                m = nd > rmax
                rmax = jnp.where(m, nd, rmax)
                ridx = jnp.where(m, off + lanes, ridx)
                return rmax, ridx

            rmax, ridx = chunk
            mx = jnp.max(rmax)
            return jnp.min(jnp.where(rmax == mx, ridx, BIG))

        lax.fori_loop(0, NPOINT, fps_step, far0)

        @plsc.parallel_loop(0, NCHUNK, unroll=8)
        def sb_chunk(ci):
            off = ci * L
            xv = x_v[0, pl.ds(off, L)]
            yv = x_v[1, pl.ds(off, L)]
            zv = x_v[2, pl.ds(off, L)]
            sb_v[pl.ds(off, L)] = xv * xv + yv * yv + zv * zv
            xb_v[0, pl.ds(off, L)] = _bf16_round(xv)
            xb_v[1, pl.ds(off, L)] = _bf16_round(yv)
            xb_v[2, pl.ds(off, L)] = _bf16_round(zv)

        def ball(si, _):
            fsplat = plsc.load_gather(cent_v, [jnp.full((L,), si, jnp.int32)])
            cx = plsc.load_gather(x_v, [zeros16, fsplat])
            cy = plsc.load_gather(x_v, [ones16, fsplat])
            cz = plsc.load_gather(x_v, [twos16, fsplat])
            sa = cx * cx + cy * cy + cz * cz
            cxb = plsc.load_gather(xb_v, [zeros16, fsplat])
            cyb = plsc.load_gather(xb_v, [ones16, fsplat])
            czb = plsc.load_gather(xb_v, [twos16, fsplat])

            def cond(carry):
                ci, found = carry
                return jnp.logical_and(found >= BIG, ci < NCHUNK)

            def body(carry):
                ci, found = carry
                off = ci * L
                xv = xb_v[0, pl.ds(off, L)]
                yv = xb_v[1, pl.ds(off, L)]
                zv = xb_v[2, pl.ds(off, L)]
                sbv = sb_v[pl.ds(off, L)]
                dot = cxb * xv + cyb * yv + czb * zv
                t = -2.0 * dot
                t = t + sa
                t = t + sbv
                hit = jnp.logical_not(t > RADIUS_SQ)
                cand = jnp.min(jnp.where(hit, off + lanes, BIG))
                return ci + 1, jnp.minimum(found, cand)

            _, found = lax.while_loop(cond, body, (jnp.int32(0), BIG))
            _store_scalar_i32(idx_v, si, found)
            return 0

        lax.fori_loop(0, NPOINT, ball, 0)
        pltpu.sync_copy(idx_v, out_hbm.at[b])


_mesh = plsc.VectorSubcoreMesh(core_axis_name="c", subcore_axis_name="s")

_sc_call = pl.kernel(
    _sc_body,
    out_type=jax.ShapeDtypeStruct((B, NPOINT), jnp.int32),
    mesh=_mesh,
    compiler_params=pltpu.CompilerParams(needs_layout_passes=False),
    scratch_types=[
        pltpu.VMEM((3, N), jnp.float32),
        pltpu.VMEM((N,), jnp.float32),
        pltpu.VMEM((NPOINT,), jnp.int32),
        pltpu.VMEM((N,), jnp.float32),
        pltpu.VMEM((NPOINT,), jnp.int32),
        pltpu.VMEM((3, N), jnp.float32),
    ],
)


def kernel(xyz, cls_label, npoint):
    del cls_label, npoint
    out = _sc_call(xyz)
    return out.reshape(B, NPOINT, 1)


# speedup vs baseline: 31.5008x; 1.5078x over previous
import jax
import jax.numpy as jnp
from jax import lax
from jax.experimental import pallas as pl
from jax.experimental.pallas import tpu as pltpu
from jax.experimental.pallas import tpu_sc as plsc

L = 16
B = 8
N = 4096
NPOINT = 512
QUARTER = N // 4
QCHUNK = QUARTER // L
NCHUNK = N // L
SPOINT = NPOINT // 4
RADIUS_SQ = 0.25
BIG = 2 ** 30
SH_ROWS = 48
SH_BASE = 32


def _bf16_round(x):
    u = plsc.bitcast(x, jnp.uint32)
    u = (u + jnp.uint32(0x7FFF) + ((u >> 16) & jnp.uint32(1))) & jnp.uint32(0xFFFF0000)
    return plsc.bitcast(u, jnp.float32)


def _store_scalar_i32(ref, pos, valv):
    lane0 = lax.iota(jnp.int32, L) == 0
    plsc.store_scatter(ref, [jnp.full((L,), pos, jnp.int32)], valv, mask=lane0)


def _sc_body(xyz_hbm, out_hbm, x_v, dist_v, cent_v, sb_v, idx_v, xb_v,
             pair_v, comb_v, sh):
    cid = lax.axis_index("c")
    sid = lax.axis_index("s")
    g = sid // 4
    q = sid % 4
    b = cid * 4 + g
    qbase = q * QUARTER
    lanes = lax.iota(jnp.int32, L)

    pltpu.sync_copy(xyz_hbm.at[b], x_v)

    zeros16 = jnp.zeros((L,), jnp.int32)
    ones16 = jnp.full((L,), 1, jnp.int32)
    twos16 = jnp.full((L,), 2, jnp.int32)

    def combine(lmaxv, lidxv):
        mx = jnp.max(lmaxv)
        li = jnp.min(jnp.where(lmaxv == mx, lidxv, BIG))
        pair_v[0, :] = jnp.full((L,), mx, jnp.float32)
        pair_v[1, :] = plsc.bitcast(jnp.full((L,), li, jnp.int32), jnp.float32)
        pltpu.sync_copy(pair_v, sh.at[SH_BASE + sid])
        plsc.subcore_barrier()
        pltpu.sync_copy(sh.at[pl.ds(SH_BASE + g * 4, 4)], comb_v)
        plsc.subcore_barrier()
        v0, i0 = comb_v[0, 0], plsc.bitcast(comb_v[0, 1], jnp.int32)
        v1, i1 = comb_v[1, 0], plsc.bitcast(comb_v[1, 1], jnp.int32)
        v2, i2 = comb_v[2, 0], plsc.bitcast(comb_v[2, 1], jnp.int32)
        v3, i3 = comb_v[3, 0], plsc.bitcast(comb_v[3, 1], jnp.int32)
        m = jnp.maximum(jnp.maximum(v0, v1), jnp.maximum(v2, v3))
        win = jnp.where(v0 == m, i0,
                        jnp.where(v1 == m, i1,
                                  jnp.where(v2 == m, i2, i3)))
        return jnp.minimum(jnp.maximum(win, 0), N - 1)

    rmax0 = jnp.full((L,), -1e30, jnp.float32)

    @plsc.parallel_loop(0, QCHUNK, unroll=8,
                        carry=(rmax0, jnp.zeros((L,), jnp.int32)))
    def init_chunk(ci, carry):
        rmax, ridx = carry
        goff = qbase + ci * L
        xv = x_v[0, pl.ds(goff, L)]
        dist_v[pl.ds(ci * L, L)] = jnp.full((L,), 1e10, jnp.float32)
        m = xv > rmax
        rmax = jnp.where(m, xv, rmax)
        ridx = jnp.where(m, goff + lanes, ridx)
        return rmax, ridx

    rmax, ridx = init_chunk
    farv0 = combine(rmax, ridx)

    def fps_step(i, farv):
        _store_scalar_i32(cent_v, i, farv)
        cx = plsc.load_gather(x_v, [zeros16, farv])
        cy = plsc.load_gather(x_v, [ones16, farv])
        cz = plsc.load_gather(x_v, [twos16, farv])

        @plsc.parallel_loop(0, QCHUNK, unroll=8,
                            carry=(jnp.full((L,), -1.0, jnp.float32),
                                   jnp.zeros((L,), jnp.int32)))
        def chunk(ci, carry):
            rmax, ridx = carry
            loff = ci * L
            goff = qbase + loff
            xv = x_v[0, pl.ds(goff, L)]
            yv = x_v[1, pl.ds(goff, L)]
            zv = x_v[2, pl.ds(goff, L)]
            dx = xv - cx
            dy = yv - cy
            dz = zv - cz
            d = dx * dx + dy * dy + dz * dz
            old = dist_v[pl.ds(loff, L)]
            nd = jnp.where(d < old, d, old)
            dist_v[pl.ds(loff, L)] = nd
            m = nd > rmax
            rmax = jnp.where(m, nd, rmax)
            ridx = jnp.where(m, goff + lanes, ridx)
            return rmax, ridx

        rmax, ridx = chunk
        return combine(rmax, ridx)

    lax.fori_loop(0, NPOINT, fps_step, farv0)

    @plsc.parallel_loop(0, NCHUNK, unroll=8)
    def sb_chunk(ci):
        off = ci * L
        xv = x_v[0, pl.ds(off, L)]
        yv = x_v[1, pl.ds(off, L)]
        zv = x_v[2, pl.ds(off, L)]
        sb_v[pl.ds(off, L)] = xv * xv + yv * yv + zv * zv
        xb_v[0, pl.ds(off, L)] = _bf16_round(xv)
        xb_v[1, pl.ds(off, L)] = _bf16_round(yv)
        xb_v[2, pl.ds(off, L)] = _bf16_round(zv)

    def ball(t, _):
        si = q * SPOINT + t
        fsplat = plsc.load_gather(cent_v, [jnp.full((L,), si, jnp.int32)])
        fsplat = jnp.minimum(jnp.maximum(fsplat, 0), N - 1)
        cx = plsc.load_gather(x_v, [zeros16, fsplat])
        cy = plsc.load_gather(x_v, [ones16, fsplat])
        cz = plsc.load_gather(x_v, [twos16, fsplat])
        sa = cx * cx + cy * cy + cz * cz
        cxb = plsc.load_gather(xb_v, [zeros16, fsplat])
        cyb = plsc.load_gather(xb_v, [ones16, fsplat])
        czb = plsc.load_gather(xb_v, [twos16, fsplat])

        def cond(carry):
            ci, found = carry
            return jnp.logical_and(found >= BIG, ci < NCHUNK)

        def body(carry):
            ci, found = carry
            off = ci * L
            xv = xb_v[0, pl.ds(off, L)]
            yv = xb_v[1, pl.ds(off, L)]
            zv = xb_v[2, pl.ds(off, L)]
            sbv = sb_v[pl.ds(off, L)]
            dot = cxb * xv + cyb * yv + czb * zv
            t2 = -2.0 * dot
            t2 = t2 + sa
            t2 = t2 + sbv
            hit = jnp.logical_not(t2 > RADIUS_SQ)
            cand = jnp.min(jnp.where(hit, off + lanes, BIG))
            return ci + 1, jnp.minimum(found, cand)

        _, found = lax.while_loop(cond, body, (jnp.int32(0), jnp.int32(BIG)))
        _store_scalar_i32(idx_v, t, jnp.full((L,), found, jnp.int32))
        return 0

    lax.fori_loop(0, SPOINT, ball, 0)
    pltpu.sync_copy(idx_v, out_hbm.at[b, pl.ds(q * SPOINT, SPOINT)])


_mesh = plsc.VectorSubcoreMesh(core_axis_name="c", subcore_axis_name="s")

_sc_call = pl.kernel(
    _sc_body,
    out_type=jax.ShapeDtypeStruct((B, NPOINT), jnp.int32),
    mesh=_mesh,
    compiler_params=pltpu.CompilerParams(needs_layout_passes=False),
    scratch_types=[
        pltpu.VMEM((3, N), jnp.float32),
        pltpu.VMEM((QUARTER,), jnp.float32),
        pltpu.VMEM((NPOINT,), jnp.int32),
        pltpu.VMEM((N,), jnp.float32),
        pltpu.VMEM((SPOINT,), jnp.int32),
        pltpu.VMEM((3, N), jnp.float32),
        pltpu.VMEM((2, L), jnp.float32),
        pltpu.VMEM((4, 2, L), jnp.float32),
        pltpu.VMEM_SHARED((SH_ROWS, 2, L), jnp.float32),
    ],
)


def kernel(xyz, cls_label, npoint):
    del cls_label, npoint
    out = _sc_call(xyz)
    return out.reshape(B, NPOINT, 1)


# speedup vs baseline: 34.9873x; 1.1107x over previous
import jax
import jax.numpy as jnp
from jax import lax
from jax.experimental import pallas as pl
from jax.experimental.pallas import tpu as pltpu
from jax.experimental.pallas import tpu_sc as plsc

L = 16
B = 8
N = 4096
NPOINT = 512
QUARTER = N // 4
QCHUNK = QUARTER // L
NCHUNK = N // L
SPOINT = NPOINT // 4
RADIUS_SQ = 0.25
BIG = 2 ** 30
SH_ROWS = 64
SH_BASE = 32


def _bf16_round(x):
    u = plsc.bitcast(x, jnp.uint32)
    u = (u + jnp.uint32(0x7FFF) + ((u >> 16) & jnp.uint32(1))) & jnp.uint32(0xFFFF0000)
    return plsc.bitcast(u, jnp.float32)


def _store_scalar_i32(ref, pos, valv):
    lane0 = lax.iota(jnp.int32, L) == 0
    plsc.store_scatter(ref, [jnp.full((L,), pos, jnp.int32)], valv, mask=lane0)


def _sc_body(xyz_hbm, out_hbm, x_v, dist_v, cent_v, sb_v, idx_v, xb_v,
             pair_v, comb_v, sh):
    cid = lax.axis_index("c")
    sid = lax.axis_index("s")
    g = sid // 4
    q = sid % 4
    b = cid * 4 + g
    qbase = q * QUARTER
    lanes = lax.iota(jnp.int32, L)

    pltpu.sync_copy(xyz_hbm.at[b], x_v)

    zeros16 = jnp.zeros((L,), jnp.int32)
    ones16 = jnp.full((L,), 1, jnp.int32)
    twos16 = jnp.full((L,), 2, jnp.int32)

    def combine(lmaxv, lidxv, bank):
        mx = jnp.max(lmaxv)
        li = jnp.min(jnp.where(lmaxv == mx, lidxv, BIG))
        pair_v[0, :] = jnp.full((L,), mx, jnp.float32)
        pair_v[1, :] = plsc.bitcast(jnp.full((L,), li, jnp.int32), jnp.float32)
        base = SH_BASE + bank * 16
        pltpu.sync_copy(pair_v, sh.at[base + sid])
        plsc.subcore_barrier()
        pltpu.sync_copy(sh.at[pl.ds(base + g * 4, 4)], comb_v)
        v0, i0 = comb_v[0, 0], plsc.bitcast(comb_v[0, 1], jnp.int32)
        v1, i1 = comb_v[1, 0], plsc.bitcast(comb_v[1, 1], jnp.int32)
        v2, i2 = comb_v[2, 0], plsc.bitcast(comb_v[2, 1], jnp.int32)
        v3, i3 = comb_v[3, 0], plsc.bitcast(comb_v[3, 1], jnp.int32)
        m = jnp.maximum(jnp.maximum(v0, v1), jnp.maximum(v2, v3))
        win = jnp.where(v0 == m, i0,
                        jnp.where(v1 == m, i1,
                                  jnp.where(v2 == m, i2, i3)))
        return jnp.minimum(jnp.maximum(win, 0), N - 1)

    rmax0 = jnp.full((L,), -1e30, jnp.float32)

    @plsc.parallel_loop(0, QCHUNK, unroll=8,
                        carry=(rmax0, jnp.zeros((L,), jnp.int32)))
    def init_chunk(ci, carry):
        rmax, ridx = carry
        goff = qbase + ci * L
        xv = x_v[0, pl.ds(goff, L)]
        dist_v[pl.ds(ci * L, L)] = jnp.full((L,), 1e10, jnp.float32)
        m = xv > rmax
        rmax = jnp.where(m, xv, rmax)
        ridx = jnp.where(m, goff + lanes, ridx)
        return rmax, ridx

    rmax, ridx = init_chunk
    farv0 = combine(rmax, ridx, 0)

    def fps_step(i, farv):
        _store_scalar_i32(cent_v, i, farv)
        cx = plsc.load_gather(x_v, [zeros16, farv])
        cy = plsc.load_gather(x_v, [ones16, farv])
        cz = plsc.load_gather(x_v, [twos16, farv])

        @plsc.parallel_loop(0, QCHUNK, unroll=8,
                            carry=(jnp.full((L,), -1.0, jnp.float32),
                                   jnp.zeros((L,), jnp.int32)))
        def chunk(ci, carry):
            rmax, ridx = carry
            loff = ci * L
            goff = qbase + loff
            xv = x_v[0, pl.ds(goff, L)]
            yv = x_v[1, pl.ds(goff, L)]
            zv = x_v[2, pl.ds(goff, L)]
            dx = xv - cx
            dy = yv - cy
            dz = zv - cz
            d = dx * dx + dy * dy + dz * dz
            old = dist_v[pl.ds(loff, L)]
            nd = jnp.minimum(d, old)
            dist_v[pl.ds(loff, L)] = nd
            m = nd > rmax
            rmax = jnp.where(m, nd, rmax)
            ridx = jnp.where(m, jnp.full((L,), ci, jnp.int32), ridx)
            return rmax, ridx

        rmax, ridx = chunk
        gidx = qbase + ridx * L + lanes
        return combine(rmax, gidx, (i + 1) & 1)

    lax.fori_loop(0, NPOINT, fps_step, farv0)

    @plsc.parallel_loop(0, NCHUNK, unroll=8)
    def sb_chunk(ci):
        off = ci * L
        xv = x_v[0, pl.ds(off, L)]
        yv = x_v[1, pl.ds(off, L)]
        zv = x_v[2, pl.ds(off, L)]
        sb_v[pl.ds(off, L)] = xv * xv + yv * yv + zv * zv
        xb_v[0, pl.ds(off, L)] = _bf16_round(xv)
        xb_v[1, pl.ds(off, L)] = _bf16_round(yv)
        xb_v[2, pl.ds(off, L)] = _bf16_round(zv)

    def ball(t, _):
        si = q * SPOINT + t
        fsplat = plsc.load_gather(cent_v, [jnp.full((L,), si, jnp.int32)])
        fsplat = jnp.minimum(jnp.maximum(fsplat, 0), N - 1)
        cx = plsc.load_gather(x_v, [zeros16, fsplat])
        cy = plsc.load_gather(x_v, [ones16, fsplat])
        cz = plsc.load_gather(x_v, [twos16, fsplat])
        sa = cx * cx + cy * cy + cz * cz
        cxb = plsc.load_gather(xb_v, [zeros16, fsplat])
        cyb = plsc.load_gather(xb_v, [ones16, fsplat])
        czb = plsc.load_gather(xb_v, [twos16, fsplat])

        def cond(carry):
            ci, found = carry
            return jnp.logical_and(found >= BIG, ci < NCHUNK)

        def body(carry):
            ci, found = carry
            off = ci * L
            xv = xb_v[0, pl.ds(off, L)]
            yv = xb_v[1, pl.ds(off, L)]
            zv = xb_v[2, pl.ds(off, L)]
            sbv = sb_v[pl.ds(off, L)]
            dot = cxb * xv + cyb * yv + czb * zv
            t2 = -2.0 * dot
            t2 = t2 + sa
            t2 = t2 + sbv
            hit = jnp.logical_not(t2 > RADIUS_SQ)
            cand = jnp.min(jnp.where(hit, off + lanes, BIG))
            return ci + 1, jnp.minimum(found, cand)

        _, found = lax.while_loop(cond, body, (jnp.int32(0), jnp.int32(BIG)))
        _store_scalar_i32(idx_v, t, jnp.full((L,), found, jnp.int32))
        return 0

    lax.fori_loop(0, SPOINT, ball, 0)
    pltpu.sync_copy(idx_v, out_hbm.at[b, pl.ds(q * SPOINT, SPOINT)])


_mesh = plsc.VectorSubcoreMesh(core_axis_name="c", subcore_axis_name="s")

_sc_call = pl.kernel(
    _sc_body,
    out_type=jax.ShapeDtypeStruct((B, NPOINT), jnp.int32),
    mesh=_mesh,
    compiler_params=pltpu.CompilerParams(needs_layout_passes=False),
    scratch_types=[
        pltpu.VMEM((3, N), jnp.float32),
        pltpu.VMEM((QUARTER,), jnp.float32),
        pltpu.VMEM((NPOINT,), jnp.int32),
        pltpu.VMEM((N,), jnp.float32),
        pltpu.VMEM((SPOINT,), jnp.int32),
        pltpu.VMEM((3, N), jnp.float32),
        pltpu.VMEM((2, L), jnp.float32),
        pltpu.VMEM((4, 2, L), jnp.float32),
        pltpu.VMEM_SHARED((SH_ROWS, 2, L), jnp.float32),
    ],
)


def kernel(xyz, cls_label, npoint):
    del cls_label, npoint
    out = _sc_call(xyz)
    return out.reshape(B, NPOINT, 1)


# speedup vs baseline: 35.1803x; 1.0055x over previous
import jax
import jax.numpy as jnp
from jax import lax
from jax.experimental import pallas as pl
from jax.experimental.pallas import tpu as pltpu
from jax.experimental.pallas import tpu_sc as plsc

L = 16
B = 8
N = 4096
NPOINT = 512
QUARTER = N // 4
QCHUNK = QUARTER // L
NCHUNK = N // L
SPOINT = NPOINT // 4
RADIUS_SQ = 0.25
BIG = 2 ** 30
SH_ROWS = 64
SH_BASE = 32


def _bf16_round(x):
    u = plsc.bitcast(x, jnp.uint32)
    u = (u + jnp.uint32(0x7FFF) + ((u >> 16) & jnp.uint32(1))) & jnp.uint32(0xFFFF0000)
    return plsc.bitcast(u, jnp.float32)


def _store_scalar_i32(ref, pos, valv):
    lane0 = lax.iota(jnp.int32, L) == 0
    plsc.store_scatter(ref, [jnp.full((L,), pos, jnp.int32)], valv, mask=lane0)


def _sc_body(xyz_hbm, out_hbm, x_v, dist_v, cent_v, sb_v, idx_v, xb_v,
             pair_v, comb_v, sh):
    cid = lax.axis_index("c")
    sid = lax.axis_index("s")
    g = sid // 4
    q = sid % 4
    b = cid * 4 + g
    qbase = q * QUARTER
    lanes = lax.iota(jnp.int32, L)

    pltpu.sync_copy(xyz_hbm.at[b], x_v)

    zeros16 = jnp.zeros((L,), jnp.int32)
    ones16 = jnp.full((L,), 1, jnp.int32)
    twos16 = jnp.full((L,), 2, jnp.int32)

    def combine(lmaxv, lidxv, bank):
        mx = jnp.max(lmaxv)
        li = jnp.min(jnp.where(lmaxv == mx, lidxv, BIG))
        pair_v[0, :] = jnp.full((L,), mx, jnp.float32)
        pair_v[1, :] = plsc.bitcast(jnp.full((L,), li, jnp.int32), jnp.float32)
        base = SH_BASE + bank * 16
        pltpu.sync_copy(pair_v, sh.at[base + sid])
        plsc.subcore_barrier()
        pltpu.sync_copy(sh.at[pl.ds(base + g * 4, 4)], comb_v)
        v0, i0 = comb_v[0, 0], plsc.bitcast(comb_v[0, 1], jnp.int32)
        v1, i1 = comb_v[1, 0], plsc.bitcast(comb_v[1, 1], jnp.int32)
        v2, i2 = comb_v[2, 0], plsc.bitcast(comb_v[2, 1], jnp.int32)
        v3, i3 = comb_v[3, 0], plsc.bitcast(comb_v[3, 1], jnp.int32)
        m = jnp.maximum(jnp.maximum(v0, v1), jnp.maximum(v2, v3))
        win = jnp.where(v0 == m, i0,
                        jnp.where(v1 == m, i1,
                                  jnp.where(v2 == m, i2, i3)))
        return jnp.minimum(jnp.maximum(win, 0), N - 1)

    rmax0 = jnp.full((L,), -1e30, jnp.float32)

    @plsc.parallel_loop(0, QCHUNK, unroll=8,
                        carry=(rmax0, jnp.zeros((L,), jnp.int32)))
    def init_chunk(ci, carry):
        rmax, ridx = carry
        goff = qbase + ci * L
        xv = x_v[0, pl.ds(goff, L)]
        dist_v[pl.ds(ci * L, L)] = jnp.full((L,), 1e10, jnp.float32)
        m = xv > rmax
        rmax = jnp.where(m, xv, rmax)
        ridx = jnp.where(m, goff + lanes, ridx)
        return rmax, ridx

    rmax, ridx = init_chunk
    farv0 = combine(rmax, ridx, 0)

    def fps_step(i, farv, bank):
        _store_scalar_i32(cent_v, i, farv)
        cx = plsc.load_gather(x_v, [zeros16, farv])
        cy = plsc.load_gather(x_v, [ones16, farv])
        cz = plsc.load_gather(x_v, [twos16, farv])

        @plsc.parallel_loop(0, QCHUNK, unroll=8,
                            carry=(jnp.full((L,), -1.0, jnp.float32),
                                   jnp.zeros((L,), jnp.int32)))
        def chunk(ci, carry):
            rmax, ridx = carry
            loff = ci * L
            goff = qbase + loff
            xv = x_v[0, pl.ds(goff, L)]
            yv = x_v[1, pl.ds(goff, L)]
            zv = x_v[2, pl.ds(goff, L)]
            dx = xv - cx
            dy = yv - cy
            dz = zv - cz
            d = dx * dx + dy * dy + dz * dz
            old = dist_v[pl.ds(loff, L)]
            nd = jnp.minimum(d, old)
            dist_v[pl.ds(loff, L)] = nd
            m = nd > rmax
            rmax = jnp.where(m, nd, rmax)
            ridx = jnp.where(m, jnp.full((L,), ci, jnp.int32), ridx)
            return rmax, ridx

        rmax, ridx = chunk
        gidx = qbase + ridx * L + lanes
        return combine(rmax, gidx, bank)

    def fps_pair(j, farv):
        farv = fps_step(2 * j, farv, 1)
        return fps_step(2 * j + 1, farv, 0)

    lax.fori_loop(0, NPOINT // 2, fps_pair, farv0)

    @plsc.parallel_loop(0, NCHUNK, unroll=8)
    def sb_chunk(ci):
        off = ci * L
        xv = x_v[0, pl.ds(off, L)]
        yv = x_v[1, pl.ds(off, L)]
        zv = x_v[2, pl.ds(off, L)]
        sb_v[pl.ds(off, L)] = xv * xv + yv * yv + zv * zv
        xb_v[0, pl.ds(off, L)] = _bf16_round(xv)
        xb_v[1, pl.ds(off, L)] = _bf16_round(yv)
        xb_v[2, pl.ds(off, L)] = _bf16_round(zv)

    def ball(t, _):
        si = q * SPOINT + t
        fsplat = plsc.load_gather(cent_v, [jnp.full((L,), si, jnp.int32)])
        fsplat = jnp.minimum(jnp.maximum(fsplat, 0), N - 1)
        cx = plsc.load_gather(x_v, [zeros16, fsplat])
        cy = plsc.load_gather(x_v, [ones16, fsplat])
        cz = plsc.load_gather(x_v, [twos16, fsplat])
        sa = cx * cx + cy * cy + cz * cz
        cxb = plsc.load_gather(xb_v, [zeros16, fsplat])
        cyb = plsc.load_gather(xb_v, [ones16, fsplat])
        czb = plsc.load_gather(xb_v, [twos16, fsplat])

        def cond(carry):
            ci, found = carry
            return jnp.logical_and(found >= BIG, ci < NCHUNK)

        def body(carry):
            ci, found = carry
            off = ci * L
            xv = xb_v[0, pl.ds(off, L)]
            yv = xb_v[1, pl.ds(off, L)]
            zv = xb_v[2, pl.ds(off, L)]
            sbv = sb_v[pl.ds(off, L)]
            dot = cxb * xv + cyb * yv + czb * zv
            t2 = -2.0 * dot
            t2 = t2 + sa
            t2 = t2 + sbv
            hit = jnp.logical_not(t2 > RADIUS_SQ)
            cand = jnp.min(jnp.where(hit, off + lanes, BIG))
            return ci + 1, jnp.minimum(found, cand)

        _, found = lax.while_loop(cond, body, (jnp.int32(0), jnp.int32(BIG)))
        _store_scalar_i32(idx_v, t, jnp.full((L,), found, jnp.int32))
        return 0

    lax.fori_loop(0, SPOINT, ball, 0)
    pltpu.sync_copy(idx_v, out_hbm.at[b, pl.ds(q * SPOINT, SPOINT)])


_mesh = plsc.VectorSubcoreMesh(core_axis_name="c", subcore_axis_name="s")

_sc_call = pl.kernel(
    _sc_body,
    out_type=jax.ShapeDtypeStruct((B, NPOINT), jnp.int32),
    mesh=_mesh,
    compiler_params=pltpu.CompilerParams(needs_layout_passes=False),
    scratch_types=[
        pltpu.VMEM((3, N), jnp.float32),
        pltpu.VMEM((QUARTER,), jnp.float32),
        pltpu.VMEM((NPOINT,), jnp.int32),
        pltpu.VMEM((N,), jnp.float32),
        pltpu.VMEM((SPOINT,), jnp.int32),
        pltpu.VMEM((3, N), jnp.float32),
        pltpu.VMEM((2, L), jnp.float32),
        pltpu.VMEM((4, 2, L), jnp.float32),
        pltpu.VMEM_SHARED((SH_ROWS, 2, L), jnp.float32),
    ],
)


def kernel(xyz, cls_label, npoint):
    del cls_label, npoint
    out = _sc_call(xyz)
    return out.reshape(B, NPOINT, 1)
